# Initial kernel scaffold; baseline (speedup 1.0000x reference)
import jax
import jax.numpy as jnp
from jax.experimental import pallas as pl


def kernel(xyz, features, W1, b1, g1, be1, W2, b2, g2, be2):
    raise NotImplementedError("write your pallas kernel here")


# speedup vs baseline: 567.7488x; 567.7488x over previous
import functools

import jax
import jax.numpy as jnp
from jax import lax
from jax.experimental import pallas as pl
from jax.experimental.pallas import tpu as pltpu
from jax.experimental.pallas import tpu_sc as plsc

_B = 8
_N = 4096
_S = 1024
_K = 16
_DIN = 128
_DOUT = 256
_ROWS = _B * _S * _K
_NTOT = float(_ROWS)


def _fps_body(xyz_ref, out_ref):
    xyzs = xyz_ref[...]
    iota24 = lax.broadcasted_iota(jnp.int32, (3 * _B, _N), 1)
    iota8 = lax.broadcasted_iota(jnp.int32, (_B, _N), 1)
    iota_s = lax.broadcasted_iota(jnp.int32, (3 * _B, _S), 1)

    def step(i, carry):
        distance, f, acc = carry
        f3 = jnp.concatenate([f, f, f], axis=0)
        sel = jnp.where(iota24 == f3, xyzs, 0.0)
        csum = jnp.sum(sel, axis=1, keepdims=True)
        acc = jnp.where(iota_s == i, csum, acc)
        diff = xyzs - csum
        sq = diff * diff
        dist = (sq[0:8] + sq[8:16]) + sq[16:24]
        distance = jnp.minimum(distance, dist)
        m = jnp.max(distance, axis=1, keepdims=True)
        f_new = jnp.min(jnp.where(distance == m, iota8, _N), axis=1,
                        keepdims=True)
        return distance, f_new, acc

    dist0 = jnp.full((_B, _N), 1e10, dtype=jnp.float32)
    f0 = jnp.zeros((_B, 1), dtype=jnp.int32)
    acc0 = jnp.zeros((3 * _B, _S), dtype=jnp.float32)
    _, _, acc = lax.fori_loop(0, _S, step, (dist0, f0, acc0))
    out_ref[...] = acc


def _fps(xyz_s):
    return pl.pallas_call(
        _fps_body,
        out_shape=jax.ShapeDtypeStruct((3 * _B, _S), jnp.float32),
    )(xyz_s)


_ST = 128


def _knn_body(xyz_ref, new_ref, out_ref):
    b = pl.program_id(0)
    P = xyz_ref[0]
    Q = new_ref[0]
    d = None
    for c in range(3):
        qc = Q[:, c:c + 1]
        pc = P[c:c + 1, :]
        t = (qc - pc) * (qc - pc)
        d = t if d is None else d + t
    iota = lax.broadcasted_iota(jnp.int32, (_ST, _N), 1)
    cols = []
    for _ in range(_K):
        m = jnp.min(d, axis=1, keepdims=True)
        amin = jnp.min(jnp.where(d == m, iota, _N), axis=1, keepdims=True)
        cols.append(amin)
        d = jnp.where(iota == amin, jnp.float32(jnp.inf), d)
    idx = jnp.concatenate(cols, axis=1)
    out_ref[0] = idx + b * _N


def _knn(xyz3, new_xyz):
    return pl.pallas_call(
        _knn_body,
        grid=(_B, _S // _ST),
        in_specs=[
            pl.BlockSpec((1, 3, _N), lambda b, st: (b, 0, 0)),
            pl.BlockSpec((1, _ST, 3), lambda b, st: (b, st, 0)),
        ],
        out_specs=pl.BlockSpec((1, _ST, _K), lambda b, st: (b, st, 0)),
        out_shape=jax.ShapeDtypeStruct((_B, _S, _K), jnp.int32),
    )(xyz3, new_xyz)


_NW = 32
_CH = 128


def _sc_gather(table, flat_idx):
    rpw = _ROWS // _NW
    nch = rpw // _CH
    mesh = plsc.VectorSubcoreMesh(core_axis_name="c", subcore_axis_name="s")

    @functools.partial(
        pl.kernel,
        mesh=mesh,
        out_type=jax.ShapeDtypeStruct((_ROWS, _DIN), jnp.float32),
        scratch_types=[
            pltpu.VMEM((rpw,), jnp.int32),
            pltpu.VMEM((_CH, _DIN), jnp.float32),
            pltpu.SemaphoreType.DMA,
        ],
    )
    def gk(table_hbm, idx_hbm, out_hbm, idx_v, buf, sem):
        wid = lax.axis_index("s") * 2 + lax.axis_index("c")
        base = wid * rpw
        pltpu.sync_copy(idx_hbm.at[pl.ds(base, rpw)], idx_v)

        def chunk(ci, carry):
            pltpu.async_copy(
                table_hbm.at[idx_v.at[pl.ds(ci * _CH, _CH)]], buf, sem
            ).wait()
            pltpu.sync_copy(buf, out_hbm.at[pl.ds(base + ci * _CH, _CH)])
            return carry

        lax.fori_loop(0, nch, chunk, 0)

    return gk(table, flat_idx)


_RT = 2048
_G = _ROWS // _RT


def _stats_rows(y):
    s = jnp.sum(y, axis=0, keepdims=True)
    ss = jnp.sum(y * y, axis=0, keepdims=True)
    return jnp.concatenate(
        [s, ss, jnp.zeros((6, y.shape[1]), jnp.float32)], axis=0)


def _m1_body(g_ref, w_ref, b_ref, y_ref, st_ref):
    i = pl.program_id(0)
    y = jnp.dot(g_ref[...], w_ref[...],
                preferred_element_type=jnp.float32) + b_ref[...]
    y_ref[...] = y
    part = _stats_rows(y)

    @pl.when(i == 0)
    def _():
        st_ref[...] = part

    @pl.when(i > 0)
    def _():
        st_ref[...] = st_ref[...] + part


def _m1(gathered, w1t, b1r):
    return pl.pallas_call(
        _m1_body,
        grid=(_G,),
        in_specs=[
            pl.BlockSpec((_RT, _DIN), lambda i: (i, 0)),
            pl.BlockSpec((_DIN, _DOUT), lambda i: (0, 0)),
            pl.BlockSpec((1, _DOUT), lambda i: (0, 0)),
        ],
        out_specs=[
            pl.BlockSpec((_RT, _DOUT), lambda i: (i, 0)),
            pl.BlockSpec((8, _DOUT), lambda i: (0, 0)),
        ],
        out_shape=[
            jax.ShapeDtypeStruct((_ROWS, _DOUT), jnp.float32),
            jax.ShapeDtypeStruct((8, _DOUT), jnp.float32),
        ],
    )(gathered, w1t, b1r)


def _bn_apply(y, st, g, be):
    mean = st[0:1] / _NTOT
    var = st[1:2] / _NTOT - mean * mean
    inv = g / jnp.sqrt(var + 1e-5)
    return (y - mean) * inv + be


def _m2_body(y1_ref, st1_ref, g_ref, be_ref, w_ref, b_ref, y2_ref, st2_ref):
    i = pl.program_id(0)
    h = jax.nn.relu(_bn_apply(y1_ref[...], st1_ref[...], g_ref[...], be_ref[...]))
    y2 = jnp.dot(h, w_ref[...],
                 preferred_element_type=jnp.float32) + b_ref[...]
    y2_ref[...] = y2
    part = _stats_rows(y2)

    @pl.when(i == 0)
    def _():
        st2_ref[...] = part

    @pl.when(i > 0)
    def _():
        st2_ref[...] = st2_ref[...] + part


def _m2(y1, st1, g1r, be1r, w2t, b2r):
    return pl.pallas_call(
        _m2_body,
        grid=(_G,),
        in_specs=[
            pl.BlockSpec((_RT, _DOUT), lambda i: (i, 0)),
            pl.BlockSpec((8, _DOUT), lambda i: (0, 0)),
            pl.BlockSpec((1, _DOUT), lambda i: (0, 0)),
            pl.BlockSpec((1, _DOUT), lambda i: (0, 0)),
            pl.BlockSpec((_DOUT, _DOUT), lambda i: (0, 0)),
            pl.BlockSpec((1, _DOUT), lambda i: (0, 0)),
        ],
        out_specs=[
            pl.BlockSpec((_RT, _DOUT), lambda i: (i, 0)),
            pl.BlockSpec((8, _DOUT), lambda i: (0, 0)),
        ],
        out_shape=[
            jax.ShapeDtypeStruct((_ROWS, _DOUT), jnp.float32),
            jax.ShapeDtypeStruct((8, _DOUT), jnp.float32),
        ],
    )(y1, st1, g1r, be1r, w2t, b2r)


_GT = 128


def _m3_body(y2_ref, st_ref, g_ref, be_ref, o_ref):
    h = jax.nn.relu(_bn_apply(y2_ref[...], st_ref[...], g_ref[...], be_ref[...]))
    o_ref[...] = jnp.max(h, axis=1)


def _m3(y2v, st2, g2r, be2r):
    ng = _B * _S // _GT
    return pl.pallas_call(
        _m3_body,
        grid=(ng,),
        in_specs=[
            pl.BlockSpec((_GT, _K, _DOUT), lambda i: (i, 0, 0)),
            pl.BlockSpec((8, _DOUT), lambda i: (0, 0)),
            pl.BlockSpec((1, _DOUT), lambda i: (0, 0)),
            pl.BlockSpec((1, _DOUT), lambda i: (0, 0)),
        ],
        out_specs=pl.BlockSpec((_GT, _DOUT), lambda i: (i, 0)),
        out_shape=jax.ShapeDtypeStruct((_B * _S, _DOUT), jnp.float32),
    )(y2v, st2, g2r, be2r)


def kernel(xyz, features, W1, b1, g1, be1, W2, b2, g2, be2):
    xyz_s = xyz.transpose(2, 0, 1).reshape(3 * _B, _N)
    newxyz_s = _fps(xyz_s)
    new_xyz = newxyz_s.reshape(3, _B, _S).transpose(1, 2, 0)

    xyz3 = xyz.transpose(0, 2, 1)
    fidx = _knn(xyz3, new_xyz)

    gathered = _sc_gather(features.reshape(_B * _N, _DIN),
                          fidx.reshape(_ROWS))

    y1, st1 = _m1(gathered, W1.T, b1.reshape(1, _DOUT))
    y2, st2 = _m2(y1, st1, g1.reshape(1, _DOUT), be1.reshape(1, _DOUT),
                  W2.T, b2.reshape(1, _DOUT))
    feats = _m3(y2.reshape(_B * _S, _K, _DOUT), st2,
                g2.reshape(1, _DOUT), be2.reshape(1, _DOUT))
    return new_xyz, feats.reshape(_B, _S, _DOUT)
